# baseline (speedup 1.0000x reference)
import jax
import jax.numpy as jnp
from jax.experimental import pallas as pl
from jax.experimental.pallas import tpu as pltpu


def _log_softmax_body(h_ref, o_ref):
    h = h_ref[...]
    m = jnp.max(h, axis=1, keepdims=True)
    s = jnp.log(jnp.sum(jnp.exp(h - m), axis=1, keepdims=True))
    o_ref[...] = h - m - s


def _gcn_conv(x, src, dst, W, b):
    n = x.shape[0]
    loop = jnp.arange(n, dtype=src.dtype)
    s = jnp.concatenate([src, loop])
    d = jnp.concatenate([dst, loop])
    xw = x @ W
    deg = jnp.zeros((n,), dtype=xw.dtype).at[d].add(1.0)
    dinv = jnp.where(deg > 0, deg ** -0.5, 0.0)
    norm = dinv[s] * dinv[d]
    msg = xw[s] * norm[:, None]
    out = jnp.zeros((n, xw.shape[1]), dtype=xw.dtype).at[d].add(msg)
    return out + b


def kernel(x, edge_index, W1, b1, W2, b2):
    src = edge_index[0]
    dst = edge_index[1]
    h = _gcn_conv(x, src, dst, W1, b1)
    h = jax.nn.relu(h)
    h = _gcn_conv(h, src, dst, W2, b2)
    out = pl.pallas_call(
        _log_softmax_body,
        out_shape=jax.ShapeDtypeStruct(h.shape, h.dtype),
    )(h)
    return out


# speedup vs baseline: 19.8573x; 19.8573x over previous
import functools

import jax
import jax.numpy as jnp
from jax import lax
from jax.experimental import pallas as pl
from jax.experimental.pallas import tpu as pltpu
from jax.experimental.pallas import tpu_sc as plsc

N = 10000
D_IN = 128
D_HID = 16
E = 320000

NW = 32
CH = 128
CPW = 79
EP = NW * CPW * CH
NP = 10112
STRIPE = NP // 16
PAD_ROW = N


def _worker(c, s):
    return c * 16 + s


def _zero_fill(ref, nrows):
    def body(i, _):
        ref[i] = jnp.zeros((D_HID,), jnp.float32)
        return 0

    lax.fori_loop(0, nrows, body, 0)


_SC_MESH = plsc.VectorSubcoreMesh(core_axis_name="c", subcore_axis_name="s")
_SC_PARAMS = pltpu.CompilerParams(use_tc_tiling_on_sc=False)


@functools.partial(
    pl.kernel,
    out_type=jax.ShapeDtypeStruct((2, NP, D_HID), jnp.float32),
    mesh=_SC_MESH,
    scratch_types=[
        pltpu.VMEM((CH,), jnp.int32),
        pltpu.VMEM((CH, D_HID), jnp.float32),
        pltpu.VMEM((STRIPE, D_HID), jnp.float32),
        pltpu.VMEM_SHARED((NP, D_HID), jnp.float32),
    ],
    compiler_params=_SC_PARAMS,
)
def _sc_degree(dst_hbm, out_hbm, dstb, ones_v, wb_v, acc):
    c = lax.axis_index("c")
    s = lax.axis_index("s")
    base = s * STRIPE
    _zero_fill(wb_v, STRIPE)
    pltpu.sync_copy(wb_v, acc.at[pl.ds(base, STRIPE)])

    def fill_ones(i, _):
        ones_v[i] = jnp.ones((D_HID,), jnp.float32)
        return 0

    lax.fori_loop(0, CH, fill_ones, 0)
    plsc.subcore_barrier()

    ebase = _worker(c, s) * (CPW * CH)

    def chunk(i, _):
        off = ebase + i * CH
        pltpu.sync_copy(dst_hbm.at[pl.ds(off, CH)], dstb)
        pltpu.sync_copy(ones_v, acc.at[dstb], add=True)
        return 0

    lax.fori_loop(0, CPW, chunk, 0)
    plsc.subcore_barrier()
    pltpu.sync_copy(acc.at[pl.ds(base, STRIPE)], wb_v)
    pltpu.sync_copy(wb_v, out_hbm.at[c].at[pl.ds(base, STRIPE)])


@functools.partial(
    pl.kernel,
    out_type=jax.ShapeDtypeStruct((2, NP, D_HID), jnp.float32),
    mesh=_SC_MESH,
    scratch_types=[
        pltpu.VMEM((CH,), jnp.int32),
        pltpu.VMEM((CH,), jnp.int32),
        pltpu.VMEM((CH, D_HID), jnp.float32),
        pltpu.VMEM((STRIPE, D_HID), jnp.float32),
        pltpu.VMEM_SHARED((NP, D_HID), jnp.float32),
        pltpu.SemaphoreType.DMA,
    ],
    compiler_params=_SC_PARAMS,
)
def _sc_aggregate(tab_hbm, src_hbm, dst_hbm, out_hbm, srcb, dstb, rows_v, wb_v,
                  acc, sem):
    c = lax.axis_index("c")
    s = lax.axis_index("s")
    base = s * STRIPE
    _zero_fill(wb_v, STRIPE)
    pltpu.sync_copy(wb_v, acc.at[pl.ds(base, STRIPE)])
    plsc.subcore_barrier()

    ebase = _worker(c, s) * (CPW * CH)

    def chunk(i, _):
        off = ebase + i * CH
        pltpu.sync_copy(src_hbm.at[pl.ds(off, CH)], srcb)
        pltpu.sync_copy(dst_hbm.at[pl.ds(off, CH)], dstb)
        pltpu.async_copy(tab_hbm.at[srcb], rows_v, sem).wait()
        pltpu.sync_copy(rows_v, acc.at[dstb], add=True)
        return 0

    lax.fori_loop(0, CPW, chunk, 0)
    plsc.subcore_barrier()
    pltpu.sync_copy(acc.at[pl.ds(base, STRIPE)], wb_v)
    pltpu.sync_copy(wb_v, out_hbm.at[c].at[pl.ds(base, STRIPE)])


def _tc_dense1_body(x_ref, w1_ref, degp_ref, xw_ref, xs_ref, dinv_ref):
    xw = jnp.dot(x_ref[...], w1_ref[...], preferred_element_type=jnp.float32)
    deg = degp_ref[0, :, :1] + degp_ref[1, :, :1] + 1.0
    dinv = lax.rsqrt(deg)
    xw_ref[...] = xw
    xs_ref[...] = xw * dinv
    dinv_ref[...] = dinv


def _tc_dense2_body(ap_ref, xw_ref, dinv_ref, b1_ref, h_ref, hs_ref):
    a = ap_ref[0] + ap_ref[1]
    dinv = dinv_ref[...]
    h = jnp.maximum(dinv * a + dinv * dinv * xw_ref[...] + b1_ref[...], 0.0)
    rows = lax.broadcasted_iota(jnp.int32, (NP, 1), 0)
    h_ref[...] = h
    hs_ref[...] = jnp.where(rows < N, h * dinv, 0.0)


def _tc_dense3_body(ap_ref, h_ref, dinv_ref, w2_ref, b2_ref, y_ref):
    a = ap_ref[0] + ap_ref[1]
    dinv = dinv_ref[...]
    out2 = dinv * a + dinv * dinv * h_ref[...]
    w2 = w2_ref[...]
    wd = w2[:, :1] - w2[:, 1:2]
    z = jnp.sum(out2 * wd.T, axis=1, keepdims=True)
    z = z + (b2_ref[0, 0] - b2_ref[0, 1])
    sp_pos = jnp.maximum(z, 0.0) + jnp.log1p(jnp.exp(-jnp.abs(z)))
    sp_neg = sp_pos - z
    y_ref[...] = jnp.concatenate([-sp_neg, -sp_pos], axis=1)


def kernel(x, edge_index, W1, b1, W2, b2):
    src = edge_index[0]
    dst = edge_index[1]
    epad = jnp.full((EP - E,), PAD_ROW, jnp.int32)
    src_p = jnp.concatenate([src, epad])
    dst_p = jnp.concatenate([dst, epad])
    x_p = jnp.concatenate([x, jnp.zeros((NP - N, D_IN), jnp.float32)])

    degp = _sc_degree(dst_p)

    xw, xs, dinv = pl.pallas_call(
        _tc_dense1_body,
        out_shape=(
            jax.ShapeDtypeStruct((NP, D_HID), jnp.float32),
            jax.ShapeDtypeStruct((NP, D_HID), jnp.float32),
            jax.ShapeDtypeStruct((NP, 1), jnp.float32),
        ),
    )(x_p, W1, degp)

    a1 = _sc_aggregate(xs, src_p, dst_p)

    h, hs = pl.pallas_call(
        _tc_dense2_body,
        out_shape=(
            jax.ShapeDtypeStruct((NP, D_HID), jnp.float32),
            jax.ShapeDtypeStruct((NP, D_HID), jnp.float32),
        ),
    )(a1, xw, dinv, b1.reshape(1, D_HID))

    a2 = _sc_aggregate(hs, src_p, dst_p)

    y = pl.pallas_call(
        _tc_dense3_body,
        out_shape=jax.ShapeDtypeStruct((NP, 2), jnp.float32),
    )(a2, h, dinv, W2, b2.reshape(1, 2))

    return y[:N]


# speedup vs baseline: 45.4297x; 2.2878x over previous
import functools

import jax
import jax.numpy as jnp
from jax import lax
from jax.experimental import pallas as pl
from jax.experimental.pallas import tpu as pltpu
from jax.experimental.pallas import tpu_sc as plsc

N = 10000
D_IN = 128
D_HID = 16
E = 320000

NW = 32
EPW = E // NW
CH = 128
K = 3
FULL = EPW // CH
G = FULL // K
TAIL = EPW - FULL * CH
NP = 10112
STRIPE = NP // 16

_SC_MESH = plsc.VectorSubcoreMesh(core_axis_name="c", subcore_axis_name="s")
_SC_PARAMS = pltpu.CompilerParams(use_tc_tiling_on_sc=False)


def _zero_fill(ref, nrows):
    def body(i, _):
        ref[i] = jnp.zeros((D_HID,), jnp.float32)
        return 0

    lax.fori_loop(0, nrows, body, 0)


@functools.partial(
    pl.kernel,
    out_type=jax.ShapeDtypeStruct((2, NP, D_HID), jnp.float32),
    mesh=_SC_MESH,
    scratch_types=[
        [pltpu.VMEM((CH,), jnp.int32) for _ in range(2 * K)],
        pltpu.VMEM((CH, D_HID), jnp.float32),
        pltpu.VMEM((STRIPE, D_HID), jnp.float32),
        pltpu.VMEM_SHARED((NP, D_HID), jnp.float32),
        [pltpu.SemaphoreType.DMA for _ in range(2)],
        [pltpu.SemaphoreType.DMA for _ in range(K)],
    ],
    compiler_params=_SC_PARAMS,
)
def _sc_degree(dst_hbm, out_hbm, dstb, ones_v, wb_v, acc, si, ss):
    c = lax.axis_index("c")
    s = lax.axis_index("s")
    base = s * STRIPE
    _zero_fill(wb_v, STRIPE)
    pltpu.sync_copy(wb_v, acc.at[pl.ds(base, STRIPE)])

    def fill_ones(i, _):
        ones_v[i] = jnp.ones((D_HID,), jnp.float32)
        return 0

    lax.fori_loop(0, CH, fill_ones, 0)
    plsc.subcore_barrier()

    ebase = (c * 16 + s) * EPW

    def idx_src(g, b):
        return dst_hbm.at[pl.ds(ebase + (g * K + b) * CH, CH)]

    for b in range(K):
        pltpu.async_copy(idx_src(0, b), dstb[b], si[0])

    def phase(g, g2, p, first, last):
        for b in range(K):
            pltpu.make_async_copy(idx_src(g, b), dstb[p * K + b], si[p]).wait()

        def wait_scatters():
            for b in range(K):
                pltpu.make_async_copy(ones_v, acc.at[dstb[p * K + b]],
                                      ss[b]).wait()

        if first:
            pl.when(g2 > 0)(wait_scatters)
        else:
            wait_scatters()

        for b in range(K):
            pltpu.async_copy(ones_v, acc.at[dstb[p * K + b]], ss[b], add=True)

        def prefetch():
            for b in range(K):
                pltpu.async_copy(idx_src(g + 1, b), dstb[(1 - p) * K + b],
                                 si[1 - p])

        if last:
            pl.when(g2 + 1 < G // 2)(prefetch)
        else:
            prefetch()

    def step(g2, _):
        phase(2 * g2, g2, 0, True, False)
        phase(2 * g2 + 1, g2, 1, False, True)
        return 0

    lax.fori_loop(0, G // 2, step, 0)
    for b in range(K):
        pltpu.make_async_copy(ones_v, acc.at[dstb[b]], ss[b]).wait()

    pltpu.sync_copy(dst_hbm.at[pl.ds(ebase + FULL * CH, TAIL)],
                    dstb[0].at[pl.ds(0, TAIL)])
    pltpu.sync_copy(ones_v.at[pl.ds(0, TAIL)],
                    acc.at[dstb[0].at[pl.ds(0, TAIL)]], add=True)

    plsc.subcore_barrier()
    pltpu.sync_copy(acc.at[pl.ds(base, STRIPE)], wb_v)
    pltpu.sync_copy(wb_v, out_hbm.at[c].at[pl.ds(base, STRIPE)])


@functools.partial(
    pl.kernel,
    out_type=jax.ShapeDtypeStruct((2, NP, D_HID), jnp.float32),
    mesh=_SC_MESH,
    scratch_types=[
        [pltpu.VMEM((CH,), jnp.int32) for _ in range(2 * K)],
        [pltpu.VMEM((CH,), jnp.int32) for _ in range(2 * K)],
        [pltpu.VMEM((CH, D_HID), jnp.float32) for _ in range(K)],
        pltpu.VMEM((TAIL,), jnp.int32),
        pltpu.VMEM((TAIL,), jnp.int32),
        pltpu.VMEM((TAIL, D_HID), jnp.float32),
        pltpu.VMEM((STRIPE, D_HID), jnp.float32),
        pltpu.VMEM_SHARED((NP, D_HID), jnp.float32),
        [pltpu.SemaphoreType.DMA for _ in range(2)],
        [pltpu.SemaphoreType.DMA for _ in range(K)],
        [pltpu.SemaphoreType.DMA for _ in range(K)],
    ],
    compiler_params=_SC_PARAMS,
)
def _sc_aggregate(tab_hbm, src_hbm, dst_hbm, out_hbm, srcb, dstb, rows, srct, dstt,
                  rowst, wb_v, acc, si, sg, ss):
    c = lax.axis_index("c")
    s = lax.axis_index("s")
    base = s * STRIPE
    _zero_fill(wb_v, STRIPE)
    pltpu.sync_copy(wb_v, acc.at[pl.ds(base, STRIPE)])
    plsc.subcore_barrier()

    ebase = (c * 16 + s) * EPW

    def idx_at(row, g, b):
        ref = src_hbm if row == 0 else dst_hbm
        return ref.at[pl.ds(ebase + (g * K + b) * CH, CH)]

    for b in range(K):
        pltpu.async_copy(idx_at(0, 0, b), srcb[b], si[0])
        pltpu.async_copy(idx_at(1, 0, b), dstb[b], si[0])

    def phase(g, g2, p, first, last):
        for b in range(K):
            pltpu.make_async_copy(idx_at(0, g, b), srcb[p * K + b], si[p]).wait()
            pltpu.make_async_copy(idx_at(1, g, b), dstb[p * K + b], si[p]).wait()

        def wait_scatters():
            for b in range(K):
                pltpu.make_async_copy(rows[b], acc.at[dstb[p * K + b]],
                                      ss[b]).wait()

        if first:
            pl.when(g2 > 0)(wait_scatters)
        else:
            wait_scatters()

        for b in range(K):
            pltpu.async_copy(tab_hbm.at[srcb[p * K + b]], rows[b], sg[b])

        def prefetch():
            for b in range(K):
                pltpu.async_copy(idx_at(0, g + 1, b), srcb[(1 - p) * K + b],
                                 si[1 - p])
                pltpu.async_copy(idx_at(1, g + 1, b), dstb[(1 - p) * K + b],
                                 si[1 - p])

        if last:
            pl.when(g2 + 1 < G // 2)(prefetch)
        else:
            prefetch()

        for b in range(K):
            pltpu.make_async_copy(tab_hbm.at[srcb[p * K + b]], rows[b],
                                  sg[b]).wait()
            pltpu.async_copy(rows[b], acc.at[dstb[p * K + b]], ss[b], add=True)

    def step(g2, _):
        phase(2 * g2, g2, 0, True, False)
        phase(2 * g2 + 1, g2, 1, False, True)
        return 0

    lax.fori_loop(0, G // 2, step, 0)
    for b in range(K):
        pltpu.make_async_copy(rows[b], acc.at[dstb[b]], ss[b]).wait()

    pltpu.sync_copy(src_hbm.at[pl.ds(ebase + FULL * CH, TAIL)], srct)
    pltpu.sync_copy(dst_hbm.at[pl.ds(ebase + FULL * CH, TAIL)], dstt)
    pltpu.sync_copy(tab_hbm.at[srct], rowst)
    pltpu.sync_copy(rowst, acc.at[dstt], add=True)

    plsc.subcore_barrier()
    pltpu.sync_copy(acc.at[pl.ds(base, STRIPE)], wb_v)
    pltpu.sync_copy(wb_v, out_hbm.at[c].at[pl.ds(base, STRIPE)])


def _tc_dense1_body(x_ref, w1_ref, degp_ref, xw_ref, xs_ref, dinv_ref):
    xw = jnp.dot(x_ref[...], w1_ref[...], preferred_element_type=jnp.float32)
    deg = degp_ref[0, :N, :1] + degp_ref[1, :N, :1] + 1.0
    dinv = lax.rsqrt(deg)
    xw_ref[...] = xw
    xs_ref[...] = xw * dinv
    dinv_ref[...] = dinv


def _tc_dense2_body(ap_ref, xw_ref, dinv_ref, b1_ref, h_ref, hs_ref):
    a = ap_ref[0, :N] + ap_ref[1, :N]
    dinv = dinv_ref[...]
    h = jnp.maximum(dinv * a + dinv * dinv * xw_ref[...] + b1_ref[...], 0.0)
    h_ref[...] = h
    hs_ref[...] = h * dinv


def _tc_dense3_body(ap_ref, h_ref, dinv_ref, w2_ref, b2_ref, y_ref):
    a = ap_ref[0, :N] + ap_ref[1, :N]
    dinv = dinv_ref[...]
    out2 = dinv * a + dinv * dinv * h_ref[...]
    w2 = w2_ref[...]
    wd = w2[:, :1] - w2[:, 1:2]
    z = jnp.sum(out2 * wd.T, axis=1, keepdims=True)
    z = z + (b2_ref[0, 0] - b2_ref[0, 1])
    sp_pos = jnp.maximum(z, 0.0) + jnp.log1p(jnp.exp(-jnp.abs(z)))
    sp_neg = sp_pos - z
    y_ref[...] = jnp.concatenate([-sp_neg, -sp_pos], axis=1)


def kernel(x, edge_index, W1, b1, W2, b2):
    src = edge_index[0]
    dst = edge_index[1]
    degp = _sc_degree(dst)

    xw, xs, dinv = pl.pallas_call(
        _tc_dense1_body,
        out_shape=(
            jax.ShapeDtypeStruct((N, D_HID), jnp.float32),
            jax.ShapeDtypeStruct((N, D_HID), jnp.float32),
            jax.ShapeDtypeStruct((N, 1), jnp.float32),
        ),
    )(x, W1, degp)

    a1 = _sc_aggregate(xs, src, dst)

    h, hs = pl.pallas_call(
        _tc_dense2_body,
        out_shape=(
            jax.ShapeDtypeStruct((N, D_HID), jnp.float32),
            jax.ShapeDtypeStruct((N, D_HID), jnp.float32),
        ),
    )(a1, xw, dinv, b1.reshape(1, D_HID))

    a2 = _sc_aggregate(hs, src, dst)

    y = pl.pallas_call(
        _tc_dense3_body,
        out_shape=jax.ShapeDtypeStruct((N, 2), jnp.float32),
    )(a2, h, dinv, W2, b2.reshape(1, 2))

    return y
